# Initial kernel scaffold; baseline (speedup 1.0000x reference)
import functools

import jax
import jax.numpy as jnp
from jax import lax
from jax.experimental import pallas as pl
from jax.experimental.pallas import tpu as pltpu
from jax.experimental.pallas import tpu_sc as plsc

N = 10000
E = 320000
HID = 256
NC = 2
NS = 16
NW = NC * NS
NPAD = 10240
SLICE = NPAD // NS
EPT = E // NW
BLK = 2000
OUTPAD = 128

_mesh = plsc.VectorSubcoreMesh(core_axis_name="c", subcore_axis_name="s")


@functools.partial(
    pl.kernel,
    out_type=jax.ShapeDtypeStruct((NC, NPAD), jnp.float32),
    mesh=_mesh,
    scratch_types=[
        pltpu.VMEM((BLK,), jnp.int32),
        pltpu.VMEM((BLK,), jnp.float32),
        pltpu.VMEM_SHARED((NPAD,), jnp.float32),
    ],
)
def _deg_kernel(dst_hbm, zeros_hbm, deg_out, idx_v, ones_v, deg_sp):
    cid = lax.axis_index("c")
    sid = lax.axis_index("s")
    sl = pl.ds(sid * SLICE, SLICE)
    pltpu.sync_copy(zeros_hbm.at[sl], deg_sp.at[sl])

    def fill(i, _):
        ones_v[pl.ds(i * 16, 16)] = jnp.ones((16,), jnp.float32)
        return 0

    lax.fori_loop(0, BLK // 16, fill, 0)
    plsc.subcore_barrier()

    base = (cid * NS + sid) * EPT

    def body(j, _):
        pltpu.sync_copy(dst_hbm.at[pl.ds(base + j * BLK, BLK)], idx_v)
        pltpu.sync_copy(ones_v, deg_sp.at[idx_v], add=True)
        return 0

    lax.fori_loop(0, EPT // BLK, body, 0)
    plsc.subcore_barrier()
    pltpu.sync_copy(deg_sp.at[sl], deg_out.at[cid, sl])


def _scale_body(deg0, deg1, xa, xb, dinv_o, x2a_o, x2b_o):
    deg = deg0[...] + deg1[...] + 1.0
    dinv = lax.rsqrt(deg)
    dinv_o[...] = dinv
    x2a_o[...] = xa[...] * dinv
    x2b_o[...] = xb[...] * dinv


_scale_kernel = pl.pallas_call(
    _scale_body,
    out_shape=[jax.ShapeDtypeStruct((NPAD // 128, 128), jnp.float32)] * 3,
)


@functools.partial(
    pl.kernel,
    out_type=jax.ShapeDtypeStruct((NC, 2, NPAD), jnp.float32),
    mesh=_mesh,
    scratch_types=[
        pltpu.VMEM((NPAD,), jnp.float32),
        pltpu.VMEM((NPAD,), jnp.float32),
        pltpu.VMEM((BLK,), jnp.int32),
        pltpu.VMEM((BLK,), jnp.int32),
        pltpu.VMEM((BLK,), jnp.float32),
        pltpu.VMEM((BLK,), jnp.float32),
        pltpu.VMEM_SHARED((NPAD,), jnp.float32),
        pltpu.VMEM_SHARED((NPAD,), jnp.float32),
    ],
)
def _agg_kernel(src_hbm, dst_hbm, x2a_hbm, x2b_hbm, zeros_hbm, acc_out,
                x2a_v, x2b_v, src_v, dst_v, va_v, vb_v, acc_a_sp, acc_b_sp):
    cid = lax.axis_index("c")
    sid = lax.axis_index("s")
    sl = pl.ds(sid * SLICE, SLICE)
    pltpu.sync_copy(x2a_hbm, x2a_v)
    pltpu.sync_copy(x2b_hbm, x2b_v)
    pltpu.sync_copy(zeros_hbm.at[sl], acc_a_sp.at[sl])
    pltpu.sync_copy(zeros_hbm.at[sl], acc_b_sp.at[sl])
    plsc.subcore_barrier()

    base = (cid * NS + sid) * EPT

    def body(j, _):
        pltpu.sync_copy(src_hbm.at[pl.ds(base + j * BLK, BLK)], src_v)
        pltpu.sync_copy(dst_hbm.at[pl.ds(base + j * BLK, BLK)], dst_v)

        def inner(i, _):
            s = src_v[pl.ds(i * 16, 16)]
            va_v[pl.ds(i * 16, 16)] = plsc.load_gather(x2a_v, [s])
            vb_v[pl.ds(i * 16, 16)] = plsc.load_gather(x2b_v, [s])
            return 0

        lax.fori_loop(0, BLK // 16, inner, 0)
        pltpu.sync_copy(va_v, acc_a_sp.at[dst_v], add=True)
        pltpu.sync_copy(vb_v, acc_b_sp.at[dst_v], add=True)
        return 0

    lax.fori_loop(0, EPT // BLK, body, 0)
    plsc.subcore_barrier()
    pltpu.sync_copy(acc_a_sp.at[sl], acc_out.at[cid, 0, sl])
    pltpu.sync_copy(acc_b_sp.at[sl], acc_out.at[cid, 1, sl])


def _final_body(acc, x2a, x2b, dinv, gw, gb, lnw, lnb, hw, hb, ow, ob, out_ref):
    CH = 1024

    def chunk(i, pooled):
        sl = pl.ds(i * CH, CH)
        aa = acc[0, 0, sl] + acc[1, 0, sl] + x2a[sl]
        bb = acc[0, 1, sl] + acc[1, 1, sl] + x2b[sl]
        dv = dinv[sl]
        g = ((aa * dv)[:, None] * gw[0][None, :]
             + (bb * dv)[:, None] * gw[1][None, :] + gb[...])
        g = jnp.maximum(g, 0.0)
        mean = jnp.mean(g, axis=1, keepdims=True)
        cen = g - mean
        var = jnp.mean(cen * cen, axis=1, keepdims=True)
        xln = cen * lax.rsqrt(var + 1e-5) * lnw[...] + lnb[...]
        rows = i * CH + lax.broadcasted_iota(jnp.int32, (CH, 1), 0)
        xln = jnp.where(rows < N, xln, 0.0)
        return pooled + jnp.sum(xln, axis=0, keepdims=True)

    pooled = lax.fori_loop(0, NPAD // CH, chunk,
                           jnp.zeros((1, HID), jnp.float32))
    h = jnp.maximum(
        jnp.dot(pooled, hw[...], preferred_element_type=jnp.float32) + hb[...],
        0.0)
    logits = jnp.dot(h, ow[...], preferred_element_type=jnp.float32) + ob[...]
    col = lax.broadcasted_iota(jnp.int32, (1, OUTPAD), 1)
    logits = jnp.where(col < 10, logits, -jnp.inf)
    m = jnp.max(logits, axis=1, keepdims=True)
    lse = jnp.log(jnp.sum(jnp.exp(logits - m), axis=1, keepdims=True)) + m
    out_ref[...] = logits - lse


_final_kernel = pl.pallas_call(
    _final_body,
    out_shape=jax.ShapeDtypeStruct((1, OUTPAD), jnp.float32),
)


@jax.jit
def kernel(node_features, edge_index, gcn_w, gcn_b, ln_w, ln_b,
           hid_w, hid_b, out_w, out_b):
    src = edge_index[0]
    dst = edge_index[1]
    zeros = jnp.zeros((NPAD,), jnp.float32)
    xa = jnp.pad(node_features[:, 0], (0, NPAD - N))
    xb = jnp.pad(node_features[:, 1], (0, NPAD - N))

    deg_part = _deg_kernel(dst, zeros)
    dinv, x2a, x2b = _scale_kernel(
        deg_part[0].reshape(NPAD // 128, 128),
        deg_part[1].reshape(NPAD // 128, 128),
        xa.reshape(NPAD // 128, 128),
        xb.reshape(NPAD // 128, 128),
    )
    dinv = dinv.reshape(NPAD)
    x2a = x2a.reshape(NPAD)
    x2b = x2b.reshape(NPAD)

    acc = _agg_kernel(src, dst, x2a, x2b, zeros)

    ob_pad = jnp.pad(out_b, (0, OUTPAD - 10)).reshape(1, OUTPAD)
    ow_pad = jnp.pad(out_w, ((0, 0), (0, OUTPAD - 10)))
    logits = _final_kernel(
        acc, x2a, x2b, dinv,
        gcn_w, gcn_b.reshape(1, HID), ln_w.reshape(1, HID),
        ln_b.reshape(1, HID), hid_w, hid_b.reshape(1, HID),
        ow_pad, ob_pad,
    )
    return logits[:, :10]


# speedup vs baseline: 98.3361x; 98.3361x over previous
import functools

import jax
import jax.numpy as jnp
from jax import lax
from jax.experimental import pallas as pl
from jax.experimental.pallas import tpu as pltpu
from jax.experimental.pallas import tpu_sc as plsc

N = 10000
E = 320000
HID = 256
NC = 2
NS = 16
NW = NC * NS
NPAD = 10240
SLICE = NPAD // NS
EPT = E // NW
BLK = 2000
OUTPAD = 128

_mesh = plsc.VectorSubcoreMesh(core_axis_name="c", subcore_axis_name="s")


@functools.partial(
    pl.kernel,
    out_type=jax.ShapeDtypeStruct((NC, NPAD), jnp.float32),
    mesh=_mesh,
    compiler_params=pltpu.CompilerParams(needs_layout_passes=False),
    scratch_types=[
        pltpu.VMEM((BLK,), jnp.int32),
        pltpu.VMEM((BLK,), jnp.float32),
        pltpu.VMEM_SHARED((NPAD,), jnp.float32),
    ],
)
def _deg_kernel(dst_hbm, zeros_hbm, deg_out, idx_v, ones_v, deg_sp):
    cid = lax.axis_index("c")
    sid = lax.axis_index("s")
    sl = pl.ds(sid * SLICE, SLICE)
    pltpu.sync_copy(zeros_hbm.at[sl], deg_sp.at[sl])

    def fill(i, _):
        ones_v[pl.ds(i * 16, 16)] = jnp.ones((16,), jnp.float32)
        return 0

    lax.fori_loop(0, BLK // 16, fill, 0)
    plsc.subcore_barrier()

    base = (cid * NS + sid) * EPT

    def body(j, _):
        pltpu.sync_copy(dst_hbm.at[pl.ds(base + j * BLK, BLK)], idx_v)
        pltpu.sync_copy(ones_v, deg_sp.at[idx_v], add=True)
        return 0

    lax.fori_loop(0, EPT // BLK, body, 0)
    plsc.subcore_barrier()
    pltpu.sync_copy(deg_sp.at[sl], deg_out.at[cid, sl])


def _scale_body(deg0, deg1, xa, xb, dinv_o, x2a_o, x2b_o):
    deg = deg0[...] + deg1[...] + 1.0
    dinv = lax.rsqrt(deg)
    dinv_o[...] = dinv
    x2a_o[...] = xa[...] * dinv
    x2b_o[...] = xb[...] * dinv


_scale_kernel = pl.pallas_call(
    _scale_body,
    out_shape=[jax.ShapeDtypeStruct((NPAD // 128, 128), jnp.float32)] * 3,
)


@functools.partial(
    pl.kernel,
    out_type=jax.ShapeDtypeStruct((NC, 2, NPAD), jnp.float32),
    mesh=_mesh,
    compiler_params=pltpu.CompilerParams(needs_layout_passes=False),
    scratch_types=[
        pltpu.VMEM((NPAD,), jnp.float32),
        pltpu.VMEM((NPAD,), jnp.float32),
        pltpu.VMEM((BLK,), jnp.int32),
        pltpu.VMEM((BLK,), jnp.int32),
        pltpu.VMEM((BLK,), jnp.float32),
        pltpu.VMEM((BLK,), jnp.float32),
        pltpu.VMEM_SHARED((NPAD,), jnp.float32),
        pltpu.VMEM_SHARED((NPAD,), jnp.float32),
    ],
)
def _agg_kernel(src_hbm, dst_hbm, x2a_hbm, x2b_hbm, zeros_hbm, acc_out,
                x2a_v, x2b_v, src_v, dst_v, va_v, vb_v, acc_a_sp, acc_b_sp):
    cid = lax.axis_index("c")
    sid = lax.axis_index("s")
    sl = pl.ds(sid * SLICE, SLICE)
    pltpu.sync_copy(x2a_hbm, x2a_v)
    pltpu.sync_copy(x2b_hbm, x2b_v)
    pltpu.sync_copy(zeros_hbm.at[sl], acc_a_sp.at[sl])
    pltpu.sync_copy(zeros_hbm.at[sl], acc_b_sp.at[sl])
    plsc.subcore_barrier()

    base = (cid * NS + sid) * EPT

    def body(j, _):
        pltpu.sync_copy(src_hbm.at[pl.ds(base + j * BLK, BLK)], src_v)
        pltpu.sync_copy(dst_hbm.at[pl.ds(base + j * BLK, BLK)], dst_v)

        def inner(i, _):
            s = src_v[pl.ds(i * 16, 16)]
            va_v[pl.ds(i * 16, 16)] = plsc.load_gather(x2a_v, [s])
            vb_v[pl.ds(i * 16, 16)] = plsc.load_gather(x2b_v, [s])
            return 0

        lax.fori_loop(0, BLK // 16, inner, 0)
        pltpu.sync_copy(va_v, acc_a_sp.at[dst_v], add=True)
        pltpu.sync_copy(vb_v, acc_b_sp.at[dst_v], add=True)
        return 0

    lax.fori_loop(0, EPT // BLK, body, 0)
    plsc.subcore_barrier()
    pltpu.sync_copy(acc_a_sp.at[sl], acc_out.at[cid, 0, sl])
    pltpu.sync_copy(acc_b_sp.at[sl], acc_out.at[cid, 1, sl])


def _final_body(acc, x2a, x2b, dinv, gw, gb, lnw, lnb, hw, hb, ow, ob, out_ref):
    CH = 1024

    def chunk(i, pooled):
        sl = pl.ds(i * CH, CH)
        aa = acc[0, 0, sl] + acc[1, 0, sl] + x2a[sl]
        bb = acc[0, 1, sl] + acc[1, 1, sl] + x2b[sl]
        dv = dinv[sl]
        g = ((aa * dv)[:, None] * gw[0][None, :]
             + (bb * dv)[:, None] * gw[1][None, :] + gb[...])
        g = jnp.maximum(g, 0.0)
        mean = jnp.mean(g, axis=1, keepdims=True)
        cen = g - mean
        var = jnp.mean(cen * cen, axis=1, keepdims=True)
        xln = cen * lax.rsqrt(var + 1e-5) * lnw[...] + lnb[...]
        rows = i * CH + lax.broadcasted_iota(jnp.int32, (CH, 1), 0)
        xln = jnp.where(rows < N, xln, 0.0)
        return pooled + jnp.sum(xln, axis=0, keepdims=True)

    pooled = lax.fori_loop(0, NPAD // CH, chunk,
                           jnp.zeros((1, HID), jnp.float32))
    h = jnp.maximum(
        jnp.dot(pooled, hw[...], preferred_element_type=jnp.float32) + hb[...],
        0.0)
    logits = jnp.dot(h, ow[...], preferred_element_type=jnp.float32) + ob[...]
    col = lax.broadcasted_iota(jnp.int32, (1, OUTPAD), 1)
    logits = jnp.where(col < 10, logits, -jnp.inf)
    m = jnp.max(logits, axis=1, keepdims=True)
    lse = jnp.log(jnp.sum(jnp.exp(logits - m), axis=1, keepdims=True)) + m
    out_ref[...] = logits - lse


_final_kernel = pl.pallas_call(
    _final_body,
    out_shape=jax.ShapeDtypeStruct((1, OUTPAD), jnp.float32),
)


@jax.jit
def kernel(node_features, edge_index, gcn_w, gcn_b, ln_w, ln_b,
           hid_w, hid_b, out_w, out_b):
    src = edge_index[0]
    dst = edge_index[1]
    zeros = jnp.zeros((NPAD,), jnp.float32)
    xa = jnp.pad(node_features[:, 0], (0, NPAD - N))
    xb = jnp.pad(node_features[:, 1], (0, NPAD - N))

    deg_part = _deg_kernel(dst, zeros)
    dinv, x2a, x2b = _scale_kernel(
        deg_part[0].reshape(NPAD // 128, 128),
        deg_part[1].reshape(NPAD // 128, 128),
        xa.reshape(NPAD // 128, 128),
        xb.reshape(NPAD // 128, 128),
    )
    dinv = dinv.reshape(NPAD)
    x2a = x2a.reshape(NPAD)
    x2b = x2b.reshape(NPAD)

    acc = _agg_kernel(src, dst, x2a, x2b, zeros)

    ob_pad = jnp.pad(out_b, (0, OUTPAD - 10)).reshape(1, OUTPAD)
    ow_pad = jnp.pad(out_w, ((0, 0), (0, OUTPAD - 10)))
    logits = _final_kernel(
        acc, x2a, x2b, dinv,
        gcn_w, gcn_b.reshape(1, HID), ln_w.reshape(1, HID),
        ln_b.reshape(1, HID), hid_w, hid_b.reshape(1, HID),
        ow_pad, ob_pad,
    )
    return logits[:, :10]


# speedup vs baseline: 99.2968x; 1.0098x over previous
import functools

import jax
import jax.numpy as jnp
from jax import lax
from jax.experimental import pallas as pl
from jax.experimental.pallas import tpu as pltpu
from jax.experimental.pallas import tpu_sc as plsc

N = 10000
E = 320000
HID = 256
NC = 2
NS = 16
NW = NC * NS
NPAD = 10240
SLICE = NPAD // NS
EPT = E // NW
BLK = 2000
NF = 2 * N
OUTPAD = 128

_mesh = plsc.VectorSubcoreMesh(core_axis_name="c", subcore_axis_name="s")
_sc_params = pltpu.CompilerParams(needs_layout_passes=False)


@functools.partial(
    pl.kernel,
    out_type=jax.ShapeDtypeStruct((NW * NPAD,), jnp.float32),
    mesh=_mesh,
    compiler_params=_sc_params,
    scratch_types=[
        pltpu.VMEM((BLK,), jnp.int32),
        pltpu.VMEM((NPAD,), jnp.float32),
    ],
)
def _deg_kernel(edge_hbm, zeros_hbm, deg_out, idx_v, acc_v):
    cid = lax.axis_index("c")
    sid = lax.axis_index("s")
    wid = cid * NS + sid
    pltpu.sync_copy(zeros_hbm, acc_v)
    base = wid * EPT
    ones16 = jnp.ones((16,), jnp.float32)

    def body(j, _):
        pltpu.sync_copy(edge_hbm.at[pl.ds(E + base + j * BLK, BLK)], idx_v)

        def inner(i, _):
            d = idx_v[pl.ds(i * 16, 16)]
            plsc.addupdate_scatter(acc_v, [d], ones16)
            return 0

        lax.fori_loop(0, BLK // 16, inner, 0)
        return 0

    lax.fori_loop(0, EPT // BLK, body, 0)
    pltpu.sync_copy(acc_v, deg_out.at[pl.ds(wid * NPAD, NPAD)])


def _scale_body(deg_parts, dinv_o):
    tot = jnp.sum(deg_parts[...], axis=0) + 1.0
    dinv_o[...] = lax.rsqrt(tot)


_scale_kernel = pl.pallas_call(
    _scale_body,
    out_shape=jax.ShapeDtypeStruct((NPAD // 128, 128), jnp.float32),
)


@functools.partial(
    pl.kernel,
    out_type=jax.ShapeDtypeStruct((NW * 2 * NPAD,), jnp.float32),
    mesh=_mesh,
    compiler_params=_sc_params,
    scratch_types=[
        pltpu.VMEM((NF,), jnp.float32),
        pltpu.VMEM((NPAD,), jnp.float32),
        pltpu.VMEM((NPAD,), jnp.float32),
        pltpu.VMEM((NPAD,), jnp.float32),
        pltpu.VMEM((BLK,), jnp.int32),
        pltpu.VMEM((BLK,), jnp.int32),
    ],
)
def _agg_kernel(edge_hbm, flat_hbm, dinv_hbm, zeros_hbm, acc_out,
                flat_v, dinv_v, acc_a, acc_b, src_v, dst_v):
    cid = lax.axis_index("c")
    sid = lax.axis_index("s")
    wid = cid * NS + sid
    pltpu.sync_copy(flat_hbm, flat_v)
    pltpu.sync_copy(dinv_hbm, dinv_v)
    pltpu.sync_copy(zeros_hbm, acc_a)
    pltpu.sync_copy(zeros_hbm, acc_b)

    iota16 = jax.lax.iota(jnp.int32, 16)

    @pl.when(cid == 0)
    def _selfloop():
        nbase = sid * SLICE

        def sbody(i, _):
            off = nbase + i * 16
            idx = off + iota16
            idxc = jnp.minimum(idx, N - 1)
            va = plsc.load_gather(flat_v, [idxc + idxc])
            vb = plsc.load_gather(flat_v, [idxc + idxc + 1])
            dv = dinv_v[pl.ds(off, 16)]
            valid = jnp.where(idx < N, dv, 0.0)
            acc_a[pl.ds(off, 16)] = va * valid
            acc_b[pl.ds(off, 16)] = vb * valid
            return 0

        lax.fori_loop(0, SLICE // 16, sbody, 0)

    base = wid * EPT

    def body(j, _):
        pltpu.sync_copy(edge_hbm.at[pl.ds(base + j * BLK, BLK)], src_v)
        pltpu.sync_copy(edge_hbm.at[pl.ds(E + base + j * BLK, BLK)], dst_v)

        def inner(i, _):
            s = src_v[pl.ds(i * 16, 16)]
            d = dst_v[pl.ds(i * 16, 16)]
            s2 = s + s
            va = plsc.load_gather(flat_v, [s2])
            vb = plsc.load_gather(flat_v, [s2 + 1])
            dv = plsc.load_gather(dinv_v, [s])
            plsc.addupdate_scatter(acc_a, [d], va * dv)
            plsc.addupdate_scatter(acc_b, [d], vb * dv)
            return 0

        lax.fori_loop(0, BLK // 16, inner, 0)
        return 0

    lax.fori_loop(0, EPT // BLK, body, 0)
    pltpu.sync_copy(acc_a, acc_out.at[pl.ds(wid * 2 * NPAD, NPAD)])
    pltpu.sync_copy(acc_b, acc_out.at[pl.ds(wid * 2 * NPAD + NPAD, NPAD)])


def _final_body(acc, dinv, gw, gb, lnw, lnb, hw, hb, ow, ob, out_ref):
    CH = 1024

    def chunk(i, pooled):
        sl = pl.ds(i * CH, CH)
        aa = jnp.sum(acc[:, 0, sl], axis=0)
        bb = jnp.sum(acc[:, 1, sl], axis=0)
        dv = dinv[sl]
        g = ((aa * dv)[:, None] * gw[0][None, :]
             + (bb * dv)[:, None] * gw[1][None, :] + gb[...])
        g = jnp.maximum(g, 0.0)
        mean = jnp.mean(g, axis=1, keepdims=True)
        cen = g - mean
        var = jnp.mean(cen * cen, axis=1, keepdims=True)
        xln = cen * lax.rsqrt(var + 1e-5) * lnw[...] + lnb[...]
        rows = i * CH + lax.broadcasted_iota(jnp.int32, (CH, 1), 0)
        xln = jnp.where(rows < N, xln, 0.0)
        return pooled + jnp.sum(xln, axis=0, keepdims=True)

    pooled = lax.fori_loop(0, NPAD // CH, chunk,
                           jnp.zeros((1, HID), jnp.float32))
    h = jnp.maximum(
        jnp.dot(pooled, hw[...], preferred_element_type=jnp.float32) + hb[...],
        0.0)
    logits = jnp.dot(h, ow[...], preferred_element_type=jnp.float32) + ob[...]
    col = lax.broadcasted_iota(jnp.int32, (1, OUTPAD), 1)
    logits = jnp.where(col < 10, logits, -jnp.inf)
    m = jnp.max(logits, axis=1, keepdims=True)
    lse = jnp.log(jnp.sum(jnp.exp(logits - m), axis=1, keepdims=True)) + m
    out_ref[...] = logits - lse


_final_kernel = pl.pallas_call(
    _final_body,
    out_shape=jax.ShapeDtypeStruct((1, OUTPAD), jnp.float32),
)


@jax.jit
def kernel(node_features, edge_index, gcn_w, gcn_b, ln_w, ln_b,
           hid_w, hid_b, out_w, out_b):
    zeros = jnp.zeros((NPAD,), jnp.float32)
    flat = node_features.reshape(NF)

    edge_flat = edge_index.reshape(2 * E)
    deg_part = _deg_kernel(edge_flat, zeros)
    dinv = _scale_kernel(deg_part.reshape(NW, NPAD // 128, 128))
    acc = _agg_kernel(edge_flat, flat, dinv.reshape(NPAD), zeros)

    ob_pad = jnp.pad(out_b, (0, OUTPAD - 10)).reshape(1, OUTPAD)
    ow_pad = jnp.pad(out_w, ((0, 0), (0, OUTPAD - 10)))
    logits = _final_kernel(
        acc.reshape(NW, 2, NPAD), dinv.reshape(NPAD),
        gcn_w, gcn_b.reshape(1, HID), ln_w.reshape(1, HID),
        ln_b.reshape(1, HID), hid_w, hid_b.reshape(1, HID),
        ow_pad, ob_pad,
    )
    return logits[:, :10]


# speedup vs baseline: 102.7201x; 1.0345x over previous
import functools

import jax
import jax.numpy as jnp
from jax import lax
from jax.experimental import pallas as pl
from jax.experimental.pallas import tpu as pltpu
from jax.experimental.pallas import tpu_sc as plsc

N = 10000
E = 320000
HID = 256
NC = 2
NS = 16
NW = NC * NS
NPAD = 10240
NPAD2 = 2 * NPAD
SLICE = NPAD // NS
EPT = E // NW
BLK = 2000
UN = 5
OUTPAD = 128

_mesh = plsc.VectorSubcoreMesh(core_axis_name="c", subcore_axis_name="s")
_sc_params = pltpu.CompilerParams(needs_layout_passes=False)


@functools.partial(
    pl.kernel,
    out_type=(
        jax.ShapeDtypeStruct((NW * NPAD,), jnp.float32),
        jax.ShapeDtypeStruct((NW * NPAD2,), jnp.float32),
    ),
    mesh=_mesh,
    compiler_params=_sc_params,
    scratch_types=[
        pltpu.VMEM((BLK,), jnp.int32),
        pltpu.VMEM((NPAD,), jnp.float32),
        pltpu.VMEM((NPAD2,), jnp.float32),
    ],
)
def _deg_kernel(edge_hbm, zeros_hbm, deg_out, deg2_out, idx_v, acc_v, acc2_v):
    cid = lax.axis_index("c")
    sid = lax.axis_index("s")
    wid = cid * NS + sid
    pltpu.sync_copy(zeros_hbm.at[pl.ds(0, NPAD)], acc_v)
    pltpu.sync_copy(zeros_hbm, acc2_v)
    base = wid * EPT
    ones16 = jnp.ones((16,), jnp.float32)

    def body(j, _):
        pltpu.sync_copy(edge_hbm.at[pl.ds(E + base + j * BLK, BLK)], idx_v)

        def inner(i, _):
            for u in range(UN):
                d = idx_v[pl.ds((i * UN + u) * 16, 16)]
                d2 = d + d
                plsc.addupdate_scatter(acc_v, [d], ones16)
                plsc.addupdate_scatter(acc2_v, [d2], ones16)
                plsc.addupdate_scatter(acc2_v, [d2 + 1], ones16)
            return 0

        lax.fori_loop(0, BLK // 16 // UN, inner, 0)
        return 0

    lax.fori_loop(0, EPT // BLK, body, 0)
    pltpu.sync_copy(acc_v, deg_out.at[pl.ds(wid * NPAD, NPAD)])
    pltpu.sync_copy(acc2_v, deg2_out.at[pl.ds(wid * NPAD2, NPAD2)])


def _scale_body(deg_parts, deg2_parts, xflat, dinv_o, x2_o):
    deg = jnp.sum(deg_parts[...].reshape(NW, NPAD // 128, 128), axis=0) + 1.0
    dinv_o[...] = 1.0 / jnp.sqrt(deg)
    deg2 = jnp.sum(deg2_parts[...].reshape(NW, NPAD2 // 128, 128), axis=0) + 1.0
    xr = xflat[...].astype(jnp.bfloat16).astype(jnp.float32)
    x2_o[...] = xr / jnp.sqrt(deg2)


_scale_kernel = pl.pallas_call(
    _scale_body,
    out_shape=[
        jax.ShapeDtypeStruct((NPAD // 128, 128), jnp.float32),
        jax.ShapeDtypeStruct((NPAD2 // 128, 128), jnp.float32),
    ],
)


@functools.partial(
    pl.kernel,
    out_type=(
        jax.ShapeDtypeStruct((NW * NPAD,), jnp.float32),
        jax.ShapeDtypeStruct((NW * NPAD,), jnp.float32),
    ),
    mesh=_mesh,
    compiler_params=_sc_params,
    scratch_types=[
        pltpu.VMEM((NPAD2,), jnp.float32),
        pltpu.VMEM((NPAD,), jnp.float32),
        pltpu.VMEM((NPAD,), jnp.float32),
        pltpu.VMEM((BLK,), jnp.int32),
        pltpu.VMEM((BLK,), jnp.int32),
    ],
)
def _agg_kernel(edge_hbm, x2_hbm, zeros_hbm, a_out, b_out,
                x2_v, acc_a, acc_b, src_v, dst_v):
    cid = lax.axis_index("c")
    sid = lax.axis_index("s")
    wid = cid * NS + sid
    pltpu.sync_copy(x2_hbm, x2_v)
    pltpu.sync_copy(zeros_hbm.at[pl.ds(0, NPAD)], acc_a)
    pltpu.sync_copy(zeros_hbm.at[pl.ds(0, NPAD)], acc_b)

    iota16 = jax.lax.iota(jnp.int32, 16)

    @pl.when(cid == 0)
    def _selfloop():
        nbase = sid * SLICE

        def sbody(i, _):
            off = nbase + i * 16
            idx2 = (off + iota16) * 2
            acc_a[pl.ds(off, 16)] = plsc.load_gather(x2_v, [idx2])
            acc_b[pl.ds(off, 16)] = plsc.load_gather(x2_v, [idx2 + 1])
            return 0

        lax.fori_loop(0, SLICE // 16, sbody, 0)

    base = wid * EPT

    def body(j, _):
        pltpu.sync_copy(edge_hbm.at[pl.ds(base + j * BLK, BLK)], src_v)
        pltpu.sync_copy(edge_hbm.at[pl.ds(E + base + j * BLK, BLK)], dst_v)

        def inner(i, _):
            for u in range(UN):
                sl = pl.ds((i * UN + u) * 16, 16)
                s = src_v[sl]
                d = dst_v[sl]
                s2 = s + s
                va = plsc.load_gather(x2_v, [s2])
                vb = plsc.load_gather(x2_v, [s2 + 1])
                plsc.addupdate_scatter(acc_a, [d], va)
                plsc.addupdate_scatter(acc_b, [d], vb)
            return 0

        lax.fori_loop(0, BLK // 16 // UN, inner, 0)
        return 0

    lax.fori_loop(0, EPT // BLK, body, 0)
    pltpu.sync_copy(acc_a, a_out.at[pl.ds(wid * NPAD, NPAD)])
    pltpu.sync_copy(acc_b, b_out.at[pl.ds(wid * NPAD, NPAD)])


def _final_body(acca, accb, dinv, gw, gb, lnw, lnb, hw, hb, ow, ob, out_ref):
    CH = 1024
    R = NPAD // 128

    A = acca[pl.ds(0, R), :]
    B = accb[pl.ds(0, R), :]
    for w in range(1, NW):
        A = A + acca[pl.ds(w * R, R), :]
        B = B + accb[pl.ds(w * R, R), :]
    DV = dinv[...]

    def _bf16r(v):
        return v.astype(jnp.bfloat16).astype(jnp.float32)

    gwr = _bf16r(gw[...])
    pooled = jnp.zeros((1, HID), jnp.float32)
    for i in range(NPAD // CH):
        aa = A[i * 8:(i + 1) * 8, :].reshape(CH)
        bb = B[i * 8:(i + 1) * 8, :].reshape(CH)
        dv = DV[i * 8:(i + 1) * 8, :].reshape(CH)
        g = ((aa * dv)[:, None] * gwr[0][None, :]
             + (bb * dv)[:, None] * gwr[1][None, :] + gb[...])
        g = jnp.maximum(g, 0.0)
        mean = jnp.mean(g, axis=1, keepdims=True)
        cen = g - mean
        var = jnp.mean(cen * cen, axis=1, keepdims=True)
        xln = cen / jnp.sqrt(var + 1e-5) * lnw[...] + lnb[...]
        rows = i * CH + lax.broadcasted_iota(jnp.int32, (CH, 1), 0)
        xln = jnp.where(rows < N, xln, 0.0)
        pooled = pooled + jnp.sum(xln, axis=0, keepdims=True)

    h = jnp.maximum(
        jnp.dot(_bf16r(pooled), _bf16r(hw[...]),
                preferred_element_type=jnp.float32) + hb[...],
        0.0)
    logits = jnp.dot(_bf16r(h), _bf16r(ow[...]),
                     preferred_element_type=jnp.float32) + ob[...]
    col = lax.broadcasted_iota(jnp.int32, (1, OUTPAD), 1)
    logits = jnp.where(col < 10, logits, -jnp.inf)
    m = jnp.max(logits, axis=1, keepdims=True)
    lse = jnp.log(jnp.sum(jnp.exp(logits - m), axis=1, keepdims=True)) + m
    out_ref[...] = logits - lse


_final_kernel = pl.pallas_call(
    _final_body,
    out_shape=jax.ShapeDtypeStruct((1, OUTPAD), jnp.float32),
)


@jax.jit
def kernel(node_features, edge_index, gcn_w, gcn_b, ln_w, ln_b,
           hid_w, hid_b, out_w, out_b):
    zeros = jnp.zeros((NPAD2,), jnp.float32)
    edge_flat = edge_index.reshape(2 * E)
    xflat = jnp.pad(node_features.reshape(2 * N), (0, NPAD2 - 2 * N))

    deg_part, deg2_part = _deg_kernel(edge_flat, zeros)
    dinv, x2 = _scale_kernel(
        deg_part.reshape(NW * NPAD // 128, 128),
        deg2_part.reshape(NW * NPAD2 // 128, 128),
        xflat.reshape(NPAD2 // 128, 128),
    )
    acc_a, acc_b = _agg_kernel(edge_flat, x2.reshape(NPAD2), zeros)

    ob_pad = jnp.pad(out_b, (0, OUTPAD - 10)).reshape(1, OUTPAD)
    ow_pad = jnp.pad(out_w, ((0, 0), (0, OUTPAD - 10)))
    logits = _final_kernel(
        acc_a.reshape(NW * NPAD // 128, 128),
        acc_b.reshape(NW * NPAD // 128, 128),
        dinv,
        gcn_w, gcn_b.reshape(1, HID), ln_w.reshape(1, HID),
        ln_b.reshape(1, HID), hid_w, hid_b.reshape(1, HID),
        ow_pad, ob_pad,
    )
    return logits[:, :10]


# speedup vs baseline: 106.2346x; 1.0342x over previous
import functools

import jax
import jax.numpy as jnp
from jax import lax
from jax.experimental import pallas as pl
from jax.experimental.pallas import tpu as pltpu
from jax.experimental.pallas import tpu_sc as plsc

N = 10000
E = 320000
HID = 256
NC = 2
NS = 16
NW = NC * NS
NP = NW + NC
NPAD = 10240
NPAD2 = 2 * NPAD
SLICE = NPAD // NS
EPT = E // NW
BLK = 2000
NBLK = EPT // BLK
UN = 5
OUTPAD = 128

_mesh = plsc.VectorSubcoreMesh(core_axis_name="c", subcore_axis_name="s")
_sc_params = pltpu.CompilerParams(needs_layout_passes=False)


@functools.partial(
    pl.kernel,
    out_type=jax.ShapeDtypeStruct((NW * NPAD,), jnp.float32),
    mesh=_mesh,
    compiler_params=_sc_params,
    scratch_types=[
        pltpu.VMEM((BLK,), jnp.int32),
        pltpu.VMEM((NPAD,), jnp.float32),
    ],
)
def _deg_kernel(edge_hbm, zeros_hbm, deg_out, idx_v, acc_v):
    cid = lax.axis_index("c")
    sid = lax.axis_index("s")
    wid = cid * NS + sid
    pltpu.sync_copy(zeros_hbm, acc_v)
    base = wid * EPT
    ones16 = jnp.ones((16,), jnp.float32)

    def body(j, _):
        pltpu.sync_copy(edge_hbm.at[pl.ds(E + base + j * BLK, BLK)], idx_v)

        def inner(i, _):
            for u in range(UN):
                d = idx_v[pl.ds((i * UN + u) * 16, 16)]
                plsc.addupdate_scatter(acc_v, [d], ones16)
            return 0

        lax.fori_loop(0, BLK // 16 // UN, inner, 0)
        return 0

    lax.fori_loop(0, NBLK, body, 0)
    pltpu.sync_copy(acc_v, deg_out.at[pl.ds(wid * NPAD, NPAD)])


def _scale_body(deg_parts, xflat, dinv_o, xr_o):
    deg = jnp.sum(deg_parts[...].reshape(NW, NPAD // 128, 128), axis=0) + 1.0
    dinv_o[...] = 1.0 / jnp.sqrt(deg)
    xr_o[...] = xflat[...].astype(jnp.bfloat16).astype(jnp.float32)


_scale_kernel = pl.pallas_call(
    _scale_body,
    out_shape=[
        jax.ShapeDtypeStruct((NPAD // 128, 128), jnp.float32),
        jax.ShapeDtypeStruct((NPAD2 // 128, 128), jnp.float32),
    ],
)


@functools.partial(
    pl.kernel,
    out_type=(
        jax.ShapeDtypeStruct((NP * NPAD,), jnp.float32),
        jax.ShapeDtypeStruct((NP * NPAD,), jnp.float32),
    ),
    mesh=_mesh,
    compiler_params=_sc_params,
    scratch_types=[
        pltpu.VMEM((NPAD2,), jnp.float32),
        pltpu.VMEM((NPAD,), jnp.float32),
        pltpu.VMEM((NPAD,), jnp.float32),
        pltpu.VMEM((NPAD,), jnp.float32),
        pltpu.VMEM((BLK,), jnp.int32),
        pltpu.VMEM((BLK,), jnp.int32),
        pltpu.VMEM((BLK,), jnp.int32),
        pltpu.VMEM((BLK,), jnp.int32),
        pltpu.VMEM((BLK,), jnp.float32),
        pltpu.VMEM((BLK,), jnp.float32),
        pltpu.VMEM_SHARED((NPAD,), jnp.float32),
        pltpu.VMEM_SHARED((NPAD,), jnp.float32),
        pltpu.SemaphoreType.DMA,
    ],
)
def _agg_kernel(edge_hbm, xr_hbm, dinv_hbm, zeros_hbm, a_out, b_out,
                xr_v, dinv_v, acc_a, acc_b, src_v, dst_v,
                ssrc_v, sdst_v, vals_a, vals_b, sacc_a, sacc_b, sem):
    cid = lax.axis_index("c")
    sid = lax.axis_index("s")
    wid = cid * NS + sid
    ssl = pl.ds(sid * SLICE, SLICE)
    pltpu.sync_copy(xr_hbm, xr_v)
    pltpu.sync_copy(dinv_hbm, dinv_v)
    pltpu.sync_copy(zeros_hbm, acc_a)
    pltpu.sync_copy(zeros_hbm, acc_b)
    pltpu.sync_copy(zeros_hbm.at[ssl], sacc_a.at[ssl])
    pltpu.sync_copy(zeros_hbm.at[ssl], sacc_b.at[ssl])
    plsc.subcore_barrier()

    base = wid * EPT

    pltpu.sync_copy(edge_hbm.at[pl.ds(base, BLK)], ssrc_v)
    pltpu.sync_copy(edge_hbm.at[pl.ds(E + base, BLK)], sdst_v)

    def sval(i, _):
        for u in range(UN):
            sl = pl.ds((i * UN + u) * 16, 16)
            s = ssrc_v[sl]
            s2 = s + s
            va = plsc.load_gather(xr_v, [s2])
            vb = plsc.load_gather(xr_v, [s2 + 1])
            dv = plsc.load_gather(dinv_v, [s])
            vals_a[sl] = va * dv
            vals_b[sl] = vb * dv
        return 0

    lax.fori_loop(0, BLK // 16 // UN, sval, 0)
    da = pltpu.async_copy(vals_a, sacc_a.at[sdst_v], sem, add=True)
    db = pltpu.async_copy(vals_b, sacc_b.at[sdst_v], sem, add=True)

    iota16 = jax.lax.iota(jnp.int32, 16)

    @pl.when(cid == 0)
    def _selfloop():
        nbase = sid * SLICE

        def sbody(i, _):
            off = nbase + i * 16
            idx2 = (off + iota16) * 2
            va = plsc.load_gather(xr_v, [idx2])
            vb = plsc.load_gather(xr_v, [idx2 + 1])
            dv = dinv_v[pl.ds(off, 16)]
            acc_a[pl.ds(off, 16)] = va * dv
            acc_b[pl.ds(off, 16)] = vb * dv
            return 0

        lax.fori_loop(0, SLICE // 16, sbody, 0)

    def body(j, _):
        pltpu.sync_copy(edge_hbm.at[pl.ds(base + j * BLK, BLK)], src_v)
        pltpu.sync_copy(edge_hbm.at[pl.ds(E + base + j * BLK, BLK)], dst_v)

        def inner(i, _):
            for u in range(UN):
                sl = pl.ds((i * UN + u) * 16, 16)
                s = src_v[sl]
                d = dst_v[sl]
                s2 = s + s
                va = plsc.load_gather(xr_v, [s2])
                vb = plsc.load_gather(xr_v, [s2 + 1])
                dv = plsc.load_gather(dinv_v, [s])
                plsc.addupdate_scatter(acc_a, [d], va * dv)
                plsc.addupdate_scatter(acc_b, [d], vb * dv)
            return 0

        lax.fori_loop(0, BLK // 16 // UN, inner, 0)
        return 0

    lax.fori_loop(1, NBLK, body, 0)

    da.wait()
    db.wait()
    plsc.subcore_barrier()

    pltpu.sync_copy(acc_a, a_out.at[pl.ds(wid * NPAD, NPAD)])
    pltpu.sync_copy(acc_b, b_out.at[pl.ds(wid * NPAD, NPAD)])

    @pl.when(sid == 0)
    def _stream_out():
        pltpu.sync_copy(sacc_a, a_out.at[pl.ds((NW + cid) * NPAD, NPAD)])
        pltpu.sync_copy(sacc_b, b_out.at[pl.ds((NW + cid) * NPAD, NPAD)])


def _final_body(acca, accb, dinv, gw, gb, lnw, lnb, hw, hb, ow, ob, out_ref):
    CH = 1024
    R = NPAD // 128

    A = acca[pl.ds(0, R), :]
    B = accb[pl.ds(0, R), :]
    for w in range(1, NP):
        A = A + acca[pl.ds(w * R, R), :]
        B = B + accb[pl.ds(w * R, R), :]
    DV = dinv[...]

    def _bf16r(v):
        return v.astype(jnp.bfloat16).astype(jnp.float32)

    gwr = _bf16r(gw[...])
    pooled = jnp.zeros((1, HID), jnp.float32)
    for i in range(NPAD // CH):
        aa = A[i * 8:(i + 1) * 8, :].reshape(CH)
        bb = B[i * 8:(i + 1) * 8, :].reshape(CH)
        dv = DV[i * 8:(i + 1) * 8, :].reshape(CH)
        g = ((aa * dv)[:, None] * gwr[0][None, :]
             + (bb * dv)[:, None] * gwr[1][None, :] + gb[...])
        g = jnp.maximum(g, 0.0)
        mean = jnp.mean(g, axis=1, keepdims=True)
        cen = g - mean
        var = jnp.mean(cen * cen, axis=1, keepdims=True)
        xln = cen / jnp.sqrt(var + 1e-5) * lnw[...] + lnb[...]
        rows = i * CH + lax.broadcasted_iota(jnp.int32, (CH, 1), 0)
        xln = jnp.where(rows < N, xln, 0.0)
        pooled = pooled + jnp.sum(xln, axis=0, keepdims=True)

    h = jnp.maximum(
        jnp.dot(_bf16r(pooled), _bf16r(hw[...]),
                preferred_element_type=jnp.float32) + hb[...],
        0.0)
    logits = jnp.dot(_bf16r(h), _bf16r(ow[...]),
                     preferred_element_type=jnp.float32) + ob[...]
    col = lax.broadcasted_iota(jnp.int32, (1, OUTPAD), 1)
    logits = jnp.where(col < 10, logits, -jnp.inf)
    m = jnp.max(logits, axis=1, keepdims=True)
    lse = jnp.log(jnp.sum(jnp.exp(logits - m), axis=1, keepdims=True)) + m
    out_ref[...] = logits - lse


_final_kernel = pl.pallas_call(
    _final_body,
    out_shape=jax.ShapeDtypeStruct((1, OUTPAD), jnp.float32),
)


@jax.jit
def kernel(node_features, edge_index, gcn_w, gcn_b, ln_w, ln_b,
           hid_w, hid_b, out_w, out_b):
    zeros = jnp.zeros((NPAD,), jnp.float32)
    edge_flat = edge_index.reshape(2 * E)
    xflat = jnp.pad(node_features.reshape(2 * N), (0, NPAD2 - 2 * N))

    deg_part = _deg_kernel(edge_flat, zeros)
    dinv, xr = _scale_kernel(
        deg_part.reshape(NW * NPAD // 128, 128),
        xflat.reshape(NPAD2 // 128, 128),
    )
    acc_a, acc_b = _agg_kernel(edge_flat, xr.reshape(NPAD2),
                               dinv.reshape(NPAD), zeros)

    ob_pad = jnp.pad(out_b, (0, OUTPAD - 10)).reshape(1, OUTPAD)
    ow_pad = jnp.pad(out_w, ((0, 0), (0, OUTPAD - 10)))
    logits = _final_kernel(
        acc_a.reshape(NP * NPAD // 128, 128),
        acc_b.reshape(NP * NPAD // 128, 128),
        dinv,
        gcn_w, gcn_b.reshape(1, HID), ln_w.reshape(1, HID),
        ln_b.reshape(1, HID), hid_w, hid_b.reshape(1, HID),
        ow_pad, ob_pad,
    )
    return logits[:, :10]


# speedup vs baseline: 108.1558x; 1.0181x over previous
import functools

import jax
import jax.numpy as jnp
from jax import lax
from jax.experimental import pallas as pl
from jax.experimental.pallas import tpu as pltpu
from jax.experimental.pallas import tpu_sc as plsc

N = 10000
E = 320000
HID = 256
NC = 2
NS = 16
NW = NC * NS
NPAD = 10240
NPAD2 = 2 * NPAD
SLICE = NPAD // NS
EPT = E // NW
BLK = 2000
NBLK = EPT // BLK
UN = 5
OUTPAD = 128

_mesh = plsc.VectorSubcoreMesh(core_axis_name="c", subcore_axis_name="s")
_sc_params = pltpu.CompilerParams(needs_layout_passes=False)


def _merge(dst_ref, src_ref):
    def body(i, _):
        for u in range(UN):
            sl = pl.ds((i * UN + u) * 16, 16)
            dst_ref[sl] = dst_ref[sl] + src_ref[sl]
        return 0

    lax.fori_loop(0, NPAD // 16 // UN, body, 0)


@functools.partial(
    pl.kernel,
    out_type=jax.ShapeDtypeStruct((NW * NPAD,), jnp.float32),
    mesh=_mesh,
    compiler_params=_sc_params,
    scratch_types=[
        pltpu.VMEM((BLK,), jnp.int32),
        pltpu.VMEM((BLK,), jnp.int32),
        pltpu.VMEM((NPAD,), jnp.float32),
        pltpu.VMEM((NPAD,), jnp.float32),
        pltpu.SemaphoreType.DMA,
        pltpu.SemaphoreType.DMA,
    ],
)
def _deg_kernel(edge_hbm, zeros_hbm, deg_out,
                idx0, idx1, acc0, acc1, sem0, sem1):
    cid = lax.axis_index("c")
    sid = lax.axis_index("s")
    wid = cid * NS + sid
    base = wid * EPT
    bufs = (idx0, idx1)
    sems = (sem0, sem1)
    ones16 = jnp.ones((16,), jnp.float32)

    d0 = pltpu.async_copy(edge_hbm.at[pl.ds(E + base, BLK)], idx0, sem0)
    pltpu.sync_copy(zeros_hbm, acc0)
    pltpu.sync_copy(zeros_hbm, acc1)
    descs = [d0]
    accs = (acc0, acc1)

    for j in range(NBLK):
        if j + 1 < NBLK:
            descs.append(pltpu.async_copy(
                edge_hbm.at[pl.ds(E + base + (j + 1) * BLK, BLK)],
                bufs[(j + 1) % 2], sems[(j + 1) % 2]))
        descs[j].wait()
        buf = bufs[j % 2]

        def inner(i, _):
            for u in range(UN):
                d = buf[pl.ds((i * UN + u) * 16, 16)]
                plsc.addupdate_scatter(accs[u % 2], [d], ones16)
            return 0

        lax.fori_loop(0, BLK // 16 // UN, inner, 0)

    _merge(acc0, acc1)
    pltpu.sync_copy(acc0, deg_out.at[pl.ds(wid * NPAD, NPAD)])


def _scale_body(deg_parts, xflat, dinv_o, xr_o):
    deg = jnp.sum(deg_parts[...].reshape(NW, NPAD // 128, 128), axis=0) + 1.0
    dinv_o[...] = 1.0 / jnp.sqrt(deg)
    xr_o[...] = xflat[...].astype(jnp.bfloat16).astype(jnp.float32)


_scale_kernel = pl.pallas_call(
    _scale_body,
    out_shape=[
        jax.ShapeDtypeStruct((NPAD // 128, 128), jnp.float32),
        jax.ShapeDtypeStruct((NPAD2 // 128, 128), jnp.float32),
    ],
)


@functools.partial(
    pl.kernel,
    out_type=(
        jax.ShapeDtypeStruct((NW * NPAD,), jnp.float32),
        jax.ShapeDtypeStruct((NW * NPAD,), jnp.float32),
    ),
    mesh=_mesh,
    compiler_params=_sc_params,
    scratch_types=[
        pltpu.VMEM((NPAD2,), jnp.float32),
        pltpu.VMEM((NPAD,), jnp.float32),
        pltpu.VMEM((NPAD,), jnp.float32),
        pltpu.VMEM((NPAD,), jnp.float32),
        pltpu.VMEM((NPAD,), jnp.float32),
        pltpu.VMEM((NPAD,), jnp.float32),
        pltpu.VMEM((BLK,), jnp.int32),
        pltpu.VMEM((BLK,), jnp.int32),
        pltpu.VMEM((BLK,), jnp.int32),
        pltpu.VMEM((BLK,), jnp.int32),
        pltpu.SemaphoreType.DMA,
        pltpu.SemaphoreType.DMA,
    ],
)
def _agg_kernel(edge_hbm, xr_hbm, dinv_hbm, zeros_hbm, a_out, b_out,
                xr_v, dinv_v, acc_a0, acc_b0, acc_a1, acc_b1,
                src0, dst0, src1, dst1, sem0, sem1):
    cid = lax.axis_index("c")
    sid = lax.axis_index("s")
    wid = cid * NS + sid
    base = wid * EPT
    sbufs = (src0, src1)
    dbufs = (dst0, dst1)
    sems = (sem0, sem1)
    accs = ((acc_a0, acc_b0), (acc_a1, acc_b1))

    descs = [(pltpu.async_copy(edge_hbm.at[pl.ds(base, BLK)], src0, sem0),
              pltpu.async_copy(edge_hbm.at[pl.ds(E + base, BLK)], dst0, sem0))]
    pltpu.sync_copy(xr_hbm, xr_v)
    pltpu.sync_copy(dinv_hbm, dinv_v)
    pltpu.sync_copy(zeros_hbm, acc_a0)
    pltpu.sync_copy(zeros_hbm, acc_b0)
    pltpu.sync_copy(zeros_hbm, acc_a1)
    pltpu.sync_copy(zeros_hbm, acc_b1)

    iota16 = jax.lax.iota(jnp.int32, 16)

    @pl.when(cid == 0)
    def _selfloop():
        nbase = sid * SLICE

        def sbody(i, _):
            off = nbase + i * 16
            idx2 = (off + iota16) * 2
            va = plsc.load_gather(xr_v, [idx2])
            vb = plsc.load_gather(xr_v, [idx2 + 1])
            dv = dinv_v[pl.ds(off, 16)]
            acc_a0[pl.ds(off, 16)] = va * dv
            acc_b0[pl.ds(off, 16)] = vb * dv
            return 0

        lax.fori_loop(0, SLICE // 16, sbody, 0)

    for j in range(NBLK):
        if j + 1 < NBLK:
            nb = (j + 1) % 2
            descs.append((
                pltpu.async_copy(
                    edge_hbm.at[pl.ds(base + (j + 1) * BLK, BLK)],
                    sbufs[nb], sems[nb]),
                pltpu.async_copy(
                    edge_hbm.at[pl.ds(E + base + (j + 1) * BLK, BLK)],
                    dbufs[nb], sems[nb])))
        descs[j][0].wait()
        descs[j][1].wait()
        sbuf = sbufs[j % 2]
        dbuf = dbufs[j % 2]

        def inner(i, _):
            for u in range(UN):
                sl = pl.ds((i * UN + u) * 16, 16)
                s = sbuf[sl]
                d = dbuf[sl]
                s2 = s + s
                va = plsc.load_gather(xr_v, [s2])
                vb = plsc.load_gather(xr_v, [s2 + 1])
                dv = plsc.load_gather(dinv_v, [s])
                aa, bb = accs[u % 2]
                plsc.addupdate_scatter(aa, [d], va * dv)
                plsc.addupdate_scatter(bb, [d], vb * dv)
            return 0

        lax.fori_loop(0, BLK // 16 // UN, inner, 0)

    _merge(acc_a0, acc_a1)
    _merge(acc_b0, acc_b1)
    pltpu.sync_copy(acc_a0, a_out.at[pl.ds(wid * NPAD, NPAD)])
    pltpu.sync_copy(acc_b0, b_out.at[pl.ds(wid * NPAD, NPAD)])


def _final_body(acca, accb, dinv, gw, gb, lnw, lnb, hw, hb, ow, ob, out_ref):
    CH = 1024
    R = NPAD // 128

    A = acca[pl.ds(0, R), :]
    B = accb[pl.ds(0, R), :]
    for w in range(1, NW):
        A = A + acca[pl.ds(w * R, R), :]
        B = B + accb[pl.ds(w * R, R), :]
    DV = dinv[...]

    def _bf16r(v):
        return v.astype(jnp.bfloat16).astype(jnp.float32)

    gwr = _bf16r(gw[...])
    pooled = jnp.zeros((1, HID), jnp.float32)
    for i in range(NPAD // CH):
        aa = A[i * 8:(i + 1) * 8, :].reshape(CH)
        bb = B[i * 8:(i + 1) * 8, :].reshape(CH)
        dv = DV[i * 8:(i + 1) * 8, :].reshape(CH)
        g = ((aa * dv)[:, None] * gwr[0][None, :]
             + (bb * dv)[:, None] * gwr[1][None, :] + gb[...])
        g = jnp.maximum(g, 0.0)
        mean = jnp.mean(g, axis=1, keepdims=True)
        cen = g - mean
        var = jnp.mean(cen * cen, axis=1, keepdims=True)
        xln = cen / jnp.sqrt(var + 1e-5) * lnw[...] + lnb[...]
        rows = i * CH + lax.broadcasted_iota(jnp.int32, (CH, 1), 0)
        xln = jnp.where(rows < N, xln, 0.0)
        pooled = pooled + jnp.sum(xln, axis=0, keepdims=True)

    h = jnp.maximum(
        jnp.dot(_bf16r(pooled), _bf16r(hw[...]),
                preferred_element_type=jnp.float32) + hb[...],
        0.0)
    logits = jnp.dot(_bf16r(h), _bf16r(ow[...]),
                     preferred_element_type=jnp.float32) + ob[...]
    col = lax.broadcasted_iota(jnp.int32, (1, OUTPAD), 1)
    logits = jnp.where(col < 10, logits, -jnp.inf)
    m = jnp.max(logits, axis=1, keepdims=True)
    lse = jnp.log(jnp.sum(jnp.exp(logits - m), axis=1, keepdims=True)) + m
    out_ref[...] = logits - lse


_final_kernel = pl.pallas_call(
    _final_body,
    out_shape=jax.ShapeDtypeStruct((1, OUTPAD), jnp.float32),
)


@jax.jit
def kernel(node_features, edge_index, gcn_w, gcn_b, ln_w, ln_b,
           hid_w, hid_b, out_w, out_b):
    zeros = jnp.zeros((NPAD,), jnp.float32)
    edge_flat = edge_index.reshape(2 * E)
    xflat = jnp.pad(node_features.reshape(2 * N), (0, NPAD2 - 2 * N))

    deg_part = _deg_kernel(edge_flat, zeros)
    dinv, xr = _scale_kernel(
        deg_part.reshape(NW * NPAD // 128, 128),
        xflat.reshape(NPAD2 // 128, 128),
    )
    acc_a, acc_b = _agg_kernel(edge_flat, xr.reshape(NPAD2),
                               dinv.reshape(NPAD), zeros)

    ob_pad = jnp.pad(out_b, (0, OUTPAD - 10)).reshape(1, OUTPAD)
    ow_pad = jnp.pad(out_w, ((0, 0), (0, OUTPAD - 10)))
    logits = _final_kernel(
        acc_a.reshape(NW * NPAD // 128, 128),
        acc_b.reshape(NW * NPAD // 128, 128),
        dinv,
        gcn_w, gcn_b.reshape(1, HID), ln_w.reshape(1, HID),
        ln_b.reshape(1, HID), hid_w, hid_b.reshape(1, HID),
        ow_pad, ob_pad,
    )
    return logits[:, :10]
